# baseline (speedup 1.0000x reference)
import functools
import jax
import jax.numpy as jnp
from jax import lax
from jax.experimental import pallas as pl
from jax.experimental.pallas import tpu as pltpu
from jax.experimental.pallas import tpu_sc as plsc

FREQ_CONST = 10000.0
NUM_CORES = 2
NUM_SUBCORES = 16
NUM_WORKERS = NUM_CORES * NUM_SUBCORES
LANES = 16


def _rope_sc(table, idx, cos_t, sin_t, *, S, D, rows_per_w, seqs_per_w):
  n_rows = idx.shape[0]
  mesh = plsc.VectorSubcoreMesh(
      core_axis_name="c", subcore_axis_name="s",
      num_cores=NUM_CORES, num_subcores=NUM_SUBCORES)
  n_chunks = D // LANES

  @functools.partial(
      pl.kernel,
      out_type=jax.ShapeDtypeStruct((n_rows, D), jnp.float32),
      mesh=mesh,
      compiler_params=pltpu.CompilerParams(use_tc_tiling_on_sc=False),
      scratch_types=dict(
          idx_v=pltpu.VMEM((rows_per_w,), jnp.int32),
          buf=pltpu.VMEM((S, D), jnp.float32),
          cos_v=pltpu.VMEM((S, D), jnp.float32),
          sin_v=pltpu.VMEM((S, D), jnp.float32),
          gsem=pltpu.SemaphoreType.DMA,
      ),
  )
  def run(table_hbm, idx_hbm, cos_hbm, sin_hbm, out_hbm, idx_v, buf, cos_v,
          sin_v, gsem):
    wid = lax.axis_index("s") * NUM_CORES + lax.axis_index("c")
    base = wid * rows_per_w
    pltpu.sync_copy(idx_hbm.at[pl.ds(base, rows_per_w)], idx_v)
    pltpu.sync_copy(cos_hbm, cos_v)
    pltpu.sync_copy(sin_hbm, sin_v)

    lane = lax.iota(jnp.int32, LANES)
    rot1 = (lane + 1) & (LANES - 1)
    zero_idx = jnp.zeros((LANES,), jnp.int32)
    last_lane = lane == (LANES - 1)

    def vgather(v, idx):
      dnums = lax.GatherDimensionNumbers(
          offset_dims=(), collapsed_slice_dims=(0,), start_index_map=(0,))
      return lax.gather(v, idx[:, None], dnums, (1,),
                        mode=lax.GatherScatterMode.PROMISE_IN_BOUNDS)

    @pl.loop(0, seqs_per_w)
    def _seq(g):
      row0 = g * S
      pltpu.async_copy(
          table_hbm.at[idx_v.at[pl.ds(row0, S)]], buf, gsem).wait()

      @pl.loop(0, S)
      def _row(s):
        e = [buf[s, pl.ds(c * LANES, LANES)] for c in range(n_chunks)]
        vals = []
        for c in range(n_chunks):
          nxt = e[(c + 1) % n_chunks]
          esh = jnp.where(last_lane, vgather(nxt, zero_idx), vgather(e[c], rot1))
          co = cos_v[s, pl.ds(c * LANES, LANES)]
          si = sin_v[s, pl.ds(c * LANES, LANES)]
          vals.append(e[c] * co + esh * si)
        for c in range(n_chunks):
          buf[s, pl.ds(c * LANES, LANES)] = vals[c]

      pltpu.sync_copy(buf, out_hbm.at[pl.ds(base + row0, S)])

  return run(table, idx, cos_t, sin_t)


def kernel(ids, token_embedding):
  B, S = ids.shape
  V, D = token_embedding.shape
  n_rows = B * S
  assert n_rows % NUM_WORKERS == 0
  rows_per_w = n_rows // NUM_WORKERS
  assert rows_per_w % S == 0
  seqs_per_w = rows_per_w // S

  ids_flat = ids.reshape(n_rows).astype(jnp.int32)
  i = jnp.arange(D, dtype=jnp.float32)
  freq = 1.0 / (FREQ_CONST ** (2.0 * jnp.floor(i / 2.0) / D))
  theta = jnp.arange(S, dtype=jnp.float32)[:, None] * freq[None, :]
  cos_t = jnp.cos(theta)
  sin_t = jnp.sin(theta)

  out = _rope_sc(token_embedding, ids_flat, cos_t, sin_t,
                 S=S, D=D, rows_per_w=rows_per_w, seqs_per_w=seqs_per_w)
  return out.reshape(B, S, D)


# speedup vs baseline: 1.0541x; 1.0541x over previous
import functools
import jax
import jax.numpy as jnp
from jax import lax
from jax.experimental import pallas as pl
from jax.experimental.pallas import tpu as pltpu
from jax.experimental.pallas import tpu_sc as plsc

FREQ_CONST = 10000.0
NUM_CORES = 2
NUM_SUBCORES = 16
NUM_WORKERS = NUM_CORES * NUM_SUBCORES
LANES = 16


def _rope_sc(table, idx, cos_t, sin_t, *, S, D, rows_per_w, seqs_per_w):
  n_rows = idx.shape[0]
  mesh = plsc.VectorSubcoreMesh(
      core_axis_name="c", subcore_axis_name="s",
      num_cores=NUM_CORES, num_subcores=NUM_SUBCORES)
  n_chunks = D // LANES

  NBUF = 4
  PREF = 2
  assert seqs_per_w % NBUF == 0 and PREF < NBUF

  @functools.partial(
      pl.kernel,
      out_type=jax.ShapeDtypeStruct((n_rows, D), jnp.float32),
      mesh=mesh,
      compiler_params=pltpu.CompilerParams(use_tc_tiling_on_sc=False),
      scratch_types=dict(
          idx_v=pltpu.VMEM((rows_per_w,), jnp.int32),
          bufs=(pltpu.VMEM((S, D), jnp.float32),) * NBUF,
          cos_v=pltpu.VMEM((S, D), jnp.float32),
          sin_v=pltpu.VMEM((S, D), jnp.float32),
          gsems=(pltpu.SemaphoreType.DMA,) * NBUF,
          osems=(pltpu.SemaphoreType.DMA,) * NBUF,
      ),
  )
  def run(table_hbm, idx_hbm, cos_hbm, sin_hbm, out_hbm, idx_v, bufs, cos_v,
          sin_v, gsems, osems):
    wid = lax.axis_index("s") * NUM_CORES + lax.axis_index("c")
    base = wid * rows_per_w
    pltpu.sync_copy(idx_hbm.at[pl.ds(base, rows_per_w)], idx_v)
    pltpu.sync_copy(cos_hbm, cos_v)
    pltpu.sync_copy(sin_hbm, sin_v)

    def gather_desc(g, b):
      return pltpu.make_async_copy(
          table_hbm.at[idx_v.at[pl.ds(g * S, S)]], bufs[b], gsems[b])

    def out_desc(g, b):
      return pltpu.make_async_copy(
          bufs[b], out_hbm.at[pl.ds(base + g * S, S)], osems[b])

    lane = lax.iota(jnp.int32, LANES)
    rot1 = (lane + 1) & (LANES - 1)
    zero_idx = jnp.zeros((LANES,), jnp.int32)
    last_lane = lane == (LANES - 1)

    def vgather(v, idx):
      dnums = lax.GatherDimensionNumbers(
          offset_dims=(), collapsed_slice_dims=(0,), start_index_map=(0,))
      return lax.gather(v, idx[:, None], dnums, (1,),
                        mode=lax.GatherScatterMode.PROMISE_IN_BOUNDS)

    def compute(buf):
      @pl.loop(0, S, unroll=4)
      def _row(s):
        e = [buf[s, pl.ds(c * LANES, LANES)] for c in range(n_chunks)]
        vals = []
        for c in range(n_chunks):
          nxt = e[(c + 1) % n_chunks]
          esh = jnp.where(last_lane, vgather(nxt, zero_idx), vgather(e[c], rot1))
          co = cos_v[s, pl.ds(c * LANES, LANES)]
          si = sin_v[s, pl.ds(c * LANES, LANES)]
          vals.append(e[c] * co + esh * si)
        for c in range(n_chunks):
          buf[s, pl.ds(c * LANES, LANES)] = vals[c]

    for b in range(PREF):
      gather_desc(b, b).start()

    @pl.loop(0, seqs_per_w, step=NBUF)
    def _ring(gg):
      for b in range(NBUF):
        g = gg + b
        gather_desc(g, b).wait()
        compute(bufs[b])
        out_desc(g, b).start()
        nb = (b + PREF) % NBUF

        @pl.when(g + PREF < seqs_per_w)
        def _():
          @pl.when(g + PREF >= NBUF)
          def _():
            out_desc(g + PREF - NBUF, nb).wait()
          gather_desc(g + PREF, nb).start()

    for b in range(NBUF):
      out_desc(seqs_per_w - NBUF + b, b).wait()

  return run(table, idx, cos_t, sin_t)


def kernel(ids, token_embedding):
  B, S = ids.shape
  V, D = token_embedding.shape
  n_rows = B * S
  assert n_rows % NUM_WORKERS == 0
  rows_per_w = n_rows // NUM_WORKERS
  assert rows_per_w % S == 0
  seqs_per_w = rows_per_w // S

  ids_flat = ids.reshape(n_rows).astype(jnp.int32)
  i = jnp.arange(D, dtype=jnp.float32)
  freq = 1.0 / (FREQ_CONST ** (2.0 * jnp.floor(i / 2.0) / D))
  theta = jnp.arange(S, dtype=jnp.float32)[:, None] * freq[None, :]
  cos_t = jnp.cos(theta)
  sin_t = jnp.sin(theta)

  out = _rope_sc(token_embedding, ids_flat, cos_t, sin_t,
                 S=S, D=D, rows_per_w=rows_per_w, seqs_per_w=seqs_per_w)
  return out.reshape(B, S, D)


# speedup vs baseline: 1.2577x; 1.1932x over previous
import functools
import jax
import jax.numpy as jnp
from jax import lax
from jax.experimental import pallas as pl
from jax.experimental.pallas import tpu as pltpu
from jax.experimental.pallas import tpu_sc as plsc

FREQ_CONST = 10000.0
NUM_CORES = 2
NUM_SUBCORES = 16
NUM_WORKERS = NUM_CORES * NUM_SUBCORES
LANES = 16
ROW_W = 128


def _rope_sc(table_p, idx, cs_t, *, S, D, rows_per_w, seqs_per_w):
  n_rows = idx.shape[0]
  mesh = plsc.VectorSubcoreMesh(
      core_axis_name="c", subcore_axis_name="s",
      num_cores=NUM_CORES, num_subcores=NUM_SUBCORES)
  n_chunks = D // LANES

  NBUF = 3
  PREF = 2

  @functools.partial(
      pl.kernel,
      out_type=jax.ShapeDtypeStruct((n_rows, ROW_W), jnp.float32),
      mesh=mesh,
      compiler_params=pltpu.CompilerParams(use_tc_tiling_on_sc=True),
      scratch_types=dict(
          idx_v=pltpu.VMEM((rows_per_w,), jnp.int32),
          bufs=(pltpu.VMEM((S, ROW_W), jnp.float32),) * NBUF,
          cs_v=pltpu.VMEM((S, ROW_W), jnp.float32),
          gsems=(pltpu.SemaphoreType.DMA,) * NBUF,
          osems=(pltpu.SemaphoreType.DMA,) * NBUF,
      ),
  )
  def run(table_hbm, idx_hbm, cs_hbm, out_hbm, idx_v, bufs,
          cs_v, gsems, osems):
    wid = lax.axis_index("s") * NUM_CORES + lax.axis_index("c")
    base = wid * rows_per_w
    pltpu.sync_copy(idx_hbm.at[pl.ds(base, rows_per_w)], idx_v)
    pltpu.sync_copy(cs_hbm, cs_v)

    def gather_desc(g, b):
      return pltpu.make_async_copy(
          table_hbm.at[idx_v.at[pl.ds(g * S, S)]], bufs[b], gsems[b])

    def out_desc(g, b):
      return pltpu.make_async_copy(
          bufs[b], out_hbm.at[pl.ds(base + g * S, S)], osems[b])

    lane = lax.iota(jnp.int32, LANES)
    rot1 = (lane + 1) & (LANES - 1)
    zero_idx = jnp.zeros((LANES,), jnp.int32)
    last_lane = lane == (LANES - 1)

    def vgather(v, idx):
      dnums = lax.GatherDimensionNumbers(
          offset_dims=(), collapsed_slice_dims=(0,), start_index_map=(0,))
      return lax.gather(v, idx[:, None], dnums, (1,),
                        mode=lax.GatherScatterMode.PROMISE_IN_BOUNDS)

    def compute(buf):
      @pl.loop(0, S, unroll=4)
      def _row(s):
        e = [buf[s, pl.ds(c * LANES, LANES)] for c in range(n_chunks)]
        vals = []
        for c in range(n_chunks):
          nxt = e[(c + 1) % n_chunks]
          esh = jnp.where(last_lane, vgather(nxt, zero_idx), vgather(e[c], rot1))
          co = cs_v[s, pl.ds(c * LANES, LANES)]
          si = cs_v[s, pl.ds(D + c * LANES, LANES)]
          vals.append(e[c] * co + esh * si)
        for c in range(n_chunks):
          buf[s, pl.ds(c * LANES, LANES)] = vals[c]

    for b in range(PREF):
      gather_desc(b, b).start()

    @pl.loop(0, seqs_per_w + (-seqs_per_w) % NBUF, step=NBUF)
    def _ring(gg):
      for b in range(NBUF):
        g = gg + b

        @pl.when(g < seqs_per_w)
        def _():
          gather_desc(g, b).wait()
          compute(bufs[b])
          out_desc(g, b).start()

          @pl.when(g + PREF < seqs_per_w)
          def _():
            @pl.when(g + PREF >= NBUF)
            def _():
              out_desc(g + PREF - NBUF, (b + PREF) % NBUF).wait()
            gather_desc(g + PREF, (b + PREF) % NBUF).start()

    for b in range(NBUF):
      g = seqs_per_w - NBUF + b
      out_desc(g, g % NBUF).wait()

  return run(table_p, idx, cs_t)


def kernel(ids, token_embedding):
  B, S = ids.shape
  V, D = token_embedding.shape
  n_rows = B * S
  assert n_rows % NUM_WORKERS == 0
  rows_per_w = n_rows // NUM_WORKERS
  assert rows_per_w % S == 0
  seqs_per_w = rows_per_w // S

  ids_flat = ids.reshape(n_rows).astype(jnp.int32)
  table_p = jnp.pad(token_embedding, ((0, 0), (0, ROW_W - D)))
  i = jnp.arange(D, dtype=jnp.float32)
  freq = 1.0 / (FREQ_CONST ** (2.0 * jnp.floor(i / 2.0) / D))
  theta = jnp.arange(S, dtype=jnp.float32)[:, None] * freq[None, :]
  cs_t = jnp.concatenate([jnp.cos(theta), jnp.sin(theta)], axis=1)

  out = _rope_sc(table_p, ids_flat, cs_t,
                 S=S, D=D, rows_per_w=rows_per_w, seqs_per_w=seqs_per_w)
  return out[:, :D].reshape(B, S, D)


# speedup vs baseline: 1.6382x; 1.3025x over previous
import functools
import jax
import jax.numpy as jnp
from jax import lax
from jax.experimental import pallas as pl
from jax.experimental.pallas import tpu as pltpu
from jax.experimental.pallas import tpu_sc as plsc

FREQ_CONST = 10000.0
NUM_CORES = 2
NUM_SUBCORES = 16
NUM_WORKERS = NUM_CORES * NUM_SUBCORES
LANES = 16
CS_W = 128


def _rope_sc(table, idx, cs_t, *, S, D, rows_per_w, seqs_per_w):
  n_rows = idx.shape[0]
  mesh = plsc.VectorSubcoreMesh(
      core_axis_name="c", subcore_axis_name="s",
      num_cores=NUM_CORES, num_subcores=NUM_SUBCORES)
  n_chunks = D // LANES

  NBUF = 3
  PREF = 2

  @functools.partial(
      pl.kernel,
      out_type=jax.ShapeDtypeStruct((n_rows, D), jnp.float32),
      mesh=mesh,
      compiler_params=pltpu.CompilerParams(use_tc_tiling_on_sc=True),
      scratch_types=dict(
          bufs=(pltpu.VMEM((S, D), jnp.float32),) * NBUF,
          idx_s=(pltpu.VMEM((S,), jnp.int32),) * NBUF,
          cs_v=pltpu.VMEM((S, CS_W), jnp.float32),
          gsems=(pltpu.SemaphoreType.DMA,) * NBUF,
          osems=(pltpu.SemaphoreType.DMA,) * NBUF,
      ),
  )
  def run(table_hbm, idx_hbm, cs_hbm, out_hbm, bufs, idx_s, cs_v,
          gsems, osems):
    wid = lax.axis_index("s") * NUM_CORES + lax.axis_index("c")
    base = wid * rows_per_w
    pltpu.sync_copy(cs_hbm, cs_v)

    def fetch_rows(g, b):
      pltpu.sync_copy(idx_hbm.at[pl.ds(base + g * S, S)], idx_s[b])

      def issue(r0, j_lo):
        vec = idx_s[b][pl.ds(r0, LANES)]
        for j in range(j_lo, LANES):
          pltpu.make_async_copy(
              table_hbm.at[pl.ds(vec[j], 1)],
              bufs[b].at[pl.ds(r0 + j, 1)], gsems[b]).start()

      n_full = S // LANES

      @pl.loop(0, n_full * LANES, step=LANES)
      def _blk(r0):
        issue(r0, 0)

      if S % LANES:
        issue(S - LANES, LANES - S % LANES)

    def fetch_drain(b):
      pltpu.make_async_copy(
          table_hbm.at[pl.ds(0, S)], bufs[b], gsems[b]).wait()

    def out_desc(g, b):
      return pltpu.make_async_copy(
          bufs[b], out_hbm.at[pl.ds(base + g * S, S)], osems[b])

    lane = lax.iota(jnp.int32, LANES)
    rot1 = (lane + 1) & (LANES - 1)
    zero_idx = jnp.zeros((LANES,), jnp.int32)
    last_lane = lane == (LANES - 1)

    def vgather(v, idx):
      dnums = lax.GatherDimensionNumbers(
          offset_dims=(), collapsed_slice_dims=(0,), start_index_map=(0,))
      return lax.gather(v, idx[:, None], dnums, (1,),
                        mode=lax.GatherScatterMode.PROMISE_IN_BOUNDS)

    def compute(buf):
      @pl.loop(0, S, unroll=4)
      def _row(s):
        e = [buf[s, pl.ds(c * LANES, LANES)] for c in range(n_chunks)]
        vals = []
        for c in range(n_chunks):
          nxt = e[(c + 1) % n_chunks]
          esh = jnp.where(last_lane, vgather(nxt, zero_idx), vgather(e[c], rot1))
          co = cs_v[s, pl.ds(c * LANES, LANES)]
          si = cs_v[s, pl.ds(D + c * LANES, LANES)]
          vals.append(e[c] * co + esh * si)
        for c in range(n_chunks):
          buf[s, pl.ds(c * LANES, LANES)] = vals[c]

    for b in range(PREF):
      fetch_rows(b, b)

    @pl.loop(0, seqs_per_w + (-seqs_per_w) % NBUF, step=NBUF)
    def _ring(gg):
      for b in range(NBUF):
        g = gg + b

        @pl.when(g < seqs_per_w)
        def _():
          fetch_drain(b)
          compute(bufs[b])
          out_desc(g, b).start()

          @pl.when(g + PREF < seqs_per_w)
          def _():
            @pl.when(g + PREF >= NBUF)
            def _():
              out_desc(g + PREF - NBUF, (b + PREF) % NBUF).wait()
            fetch_rows(g + PREF, (b + PREF) % NBUF)

    for b in range(NBUF):
      g = seqs_per_w - NBUF + b
      out_desc(g, g % NBUF).wait()

  return run(table, idx, cs_t)


def kernel(ids, token_embedding):
  B, S = ids.shape
  V, D = token_embedding.shape
  n_rows = B * S
  assert n_rows % NUM_WORKERS == 0
  rows_per_w = n_rows // NUM_WORKERS
  assert rows_per_w % S == 0
  seqs_per_w = rows_per_w // S

  ids_flat = ids.reshape(n_rows).astype(jnp.int32)
  i = jnp.arange(D, dtype=jnp.float32)
  freq = 1.0 / (FREQ_CONST ** (2.0 * jnp.floor(i / 2.0) / D))
  theta = jnp.arange(S, dtype=jnp.float32)[:, None] * freq[None, :]
  cs_t = jnp.concatenate([jnp.cos(theta), jnp.sin(theta)], axis=1)

  out = _rope_sc(token_embedding, ids_flat, cs_t,
                 S=S, D=D, rows_per_w=rows_per_w, seqs_per_w=seqs_per_w)
  return out.reshape(B, S, D)
